# Initial kernel scaffold; baseline (speedup 1.0000x reference)
import jax
import jax.numpy as jnp
from jax.experimental import pallas as pl


def kernel(x, edge_index, batch, W1, b1, W2, b2, W_out, b_out):
    raise NotImplementedError("write your pallas kernel here")


# speedup vs baseline: 27.6210x; 27.6210x over previous
import functools

import jax
import jax.numpy as jnp
from jax import lax
from jax.experimental import pallas as pl
from jax.experimental.pallas import tpu as pltpu
from jax.experimental.pallas import tpu_sc as plsc

N = 10000
E = 320000
D = 128
H1 = 16
H2 = 32
NUM_GRAPHS = 16
NUM_CLASSES = 10

NCORE = 2
NSUB = 16
NTILE = NCORE * NSUB
CH = 128
NPAD = 10240
SLICE = NPAD // NSUB
EPAD = NTILE * 80 * CH
TPW = EPAD // (NTILE * CH)
PAD_SRC = N + 8
PAD_DST = N

_mesh = plsc.VectorSubcoreMesh(core_axis_name="c", subcore_axis_name="s")
_sc_params = pltpu.CompilerParams(use_tc_tiling_on_sc=False)


@functools.partial(
    pl.kernel,
    out_type=jax.ShapeDtypeStruct((NCORE, NPAD), jnp.float32),
    mesh=_mesh,
    scratch_types=[
        pltpu.VMEM((TPW, CH), jnp.int32),
        pltpu.VMEM((CH,), jnp.float32),
        pltpu.VMEM((SLICE,), jnp.float32),
        pltpu.VMEM_SHARED((NPAD,), jnp.float32),
        pltpu.SemaphoreType.DMA,
    ],
    compiler_params=_sc_params,
)
def _deg_sc(dst_hbm, out_hbm, idx_v, ones_v, stage_v, deg_sh, sem):
    c = lax.axis_index("c")
    s = lax.axis_index("s")
    w = c * NSUB + s

    @pl.loop(0, CH // 16)
    def _(i):
        ones_v[pl.ds(i * 16, 16)] = jnp.ones((16,), jnp.float32)

    @pl.loop(0, SLICE // 16)
    def _(i):
        stage_v[pl.ds(i * 16, 16)] = jnp.zeros((16,), jnp.float32)

    pltpu.sync_copy(stage_v, deg_sh.at[pl.ds(s * SLICE, SLICE)])
    plsc.subcore_barrier()

    pltpu.async_copy(dst_hbm.at[w], idx_v, sem).wait()

    @pl.loop(0, TPW)
    def _(j):
        pltpu.sync_copy(ones_v, deg_sh.at[idx_v.at[j]], add=True)

    plsc.subcore_barrier()
    pltpu.sync_copy(deg_sh.at[pl.ds(s * SLICE, SLICE)], stage_v)
    pltpu.sync_copy(stage_v, out_hbm.at[c, pl.ds(s * SLICE, SLICE)])


def _make_prop(W):
    @functools.partial(
        pl.kernel,
        out_type=jax.ShapeDtypeStruct((NCORE, NPAD, W), jnp.float32),
        mesh=_mesh,
        scratch_types=[
            pltpu.VMEM((TPW, CH), jnp.int32),
            pltpu.VMEM((TPW, CH), jnp.int32),
            pltpu.VMEM((CH, W), jnp.float32),
            pltpu.VMEM((SLICE, W), jnp.float32),
            pltpu.VMEM_SHARED((NPAD, W), jnp.float32),
            pltpu.SemaphoreType.DMA,
        ],
        compiler_params=_sc_params,
    )
    def _prop(tab_hbm, src_hbm, dst_hbm, out_hbm, si_v, di_v, rows_v,
              stage_v, acc_sh, sem):
        c = lax.axis_index("c")
        s = lax.axis_index("s")
        w = c * NSUB + s

        @pl.loop(0, SLICE)
        def _(i):
            for k in range(W // 16):
                stage_v[i, pl.ds(k * 16, 16)] = jnp.zeros((16,), jnp.float32)

        pltpu.sync_copy(stage_v, acc_sh.at[pl.ds(s * SLICE, SLICE)])
        plsc.subcore_barrier()

        pltpu.async_copy(src_hbm.at[w], si_v, sem).wait()
        pltpu.async_copy(dst_hbm.at[w], di_v, sem).wait()

        @pl.loop(0, TPW)
        def _(j):
            pltpu.async_copy(tab_hbm.at[si_v.at[j]], rows_v, sem).wait()
            pltpu.sync_copy(rows_v, acc_sh.at[di_v.at[j]], add=True)

        plsc.subcore_barrier()
        pltpu.sync_copy(acc_sh.at[pl.ds(s * SLICE, SLICE)], stage_v)
        pltpu.sync_copy(stage_v, out_hbm.at[c, pl.ds(s * SLICE, SLICE)])

    return _prop


_prop16 = _make_prop(H1)
_prop32 = _make_prop(H2)


def _mm_body(x_ref, w_ref, o_ref):
    o_ref[...] = jnp.dot(x_ref[...], w_ref[...],
                         preferred_element_type=jnp.float32)


def _scale_body(degt_ref, h_ref, dinv_ref, hs_ref):
    deg = degt_ref[:, 0:1] + degt_ref[:, 1:2] + 1.0
    dinv = lax.rsqrt(deg)
    dinv_ref[...] = dinv
    hs_ref[...] = h_ref[...] * dinv


def _layer_body(a_ref, hs1_ref, dinv_ref, w2_ref, b1_ref, o_ref):
    acc = a_ref[0] + a_ref[1] + hs1_ref[...]
    z1 = jnp.maximum(acc * dinv_ref[...] + b1_ref[...], 0.0)
    o_ref[...] = jnp.dot(z1, w2_ref[...],
                         preferred_element_type=jnp.float32) * dinv_ref[...]


def _final_body(a_ref, hs2_ref, dinv_ref, b2_ref, batch_ref, wo_ref, bo_ref,
                o_ref):
    z2 = jnp.maximum((a_ref[0] + a_ref[1] + hs2_ref[...]) * dinv_ref[...]
                     + b2_ref[...], 0.0)
    gid = lax.broadcasted_iota(jnp.int32, (NUM_GRAPHS, NPAD), 0)
    onehot = (batch_ref[...] == gid).astype(jnp.float32)
    sums = jnp.dot(onehot, z2, preferred_element_type=jnp.float32)
    cnts = jnp.sum(onehot, axis=1, keepdims=True)
    g = sums / jnp.maximum(cnts, 1.0)
    o_ref[...] = jnp.dot(g, wo_ref[...],
                         preferred_element_type=jnp.float32) + bo_ref[...]


def _tc(body, out_shape, *args):
    return pl.pallas_call(body, out_shape=out_shape)(*args)


def kernel(x, edge_index, batch, W1, b1, W2, b2, W_out, b_out):
    f32 = jnp.float32
    src = edge_index[0]
    dst = edge_index[1]
    src_p = jnp.concatenate(
        [src, jnp.full((EPAD - E,), PAD_SRC, jnp.int32)]).reshape(NTILE, TPW, CH)
    dst_p = jnp.concatenate(
        [dst, jnp.full((EPAD - E,), PAD_DST, jnp.int32)]).reshape(NTILE, TPW, CH)
    x_p = jnp.zeros((NPAD, D), f32).at[:N].set(x)
    batch_p = jnp.concatenate(
        [batch, jnp.full((NPAD - N,), NUM_GRAPHS, jnp.int32)]).reshape(1, NPAD)

    deg2 = _deg_sc(dst_p)
    h1 = _tc(_mm_body, jax.ShapeDtypeStruct((NPAD, H1), f32), x_p, W1)
    deg_t = jnp.transpose(deg2)
    dinv, hs1 = _tc(
        _scale_body,
        (jax.ShapeDtypeStruct((NPAD, 1), f32),
         jax.ShapeDtypeStruct((NPAD, H1), f32)),
        deg_t, h1)
    acc1 = _prop16(hs1, src_p, dst_p)
    hs2 = _tc(_layer_body, jax.ShapeDtypeStruct((NPAD, H2), f32),
              acc1, hs1, dinv, W2, b1.reshape(1, H1))
    acc2 = _prop32(hs2, src_p, dst_p)
    out = _tc(_final_body,
              jax.ShapeDtypeStruct((NUM_GRAPHS, NUM_CLASSES), f32),
              acc2, hs2, dinv, b2.reshape(1, H2), batch_p, W_out,
              b_out.reshape(1, NUM_CLASSES))
    return out


# speedup vs baseline: 35.1628x; 1.2730x over previous
import functools

import jax
import jax.numpy as jnp
from jax import lax
from jax.experimental import pallas as pl
from jax.experimental.pallas import tpu as pltpu
from jax.experimental.pallas import tpu_sc as plsc

N = 10000
E = 320000
D = 128
H1 = 16
H2 = 32
NUM_GRAPHS = 16
NUM_CLASSES = 10

NCORE = 2
NSUB = 16
NTILE = NCORE * NSUB
CH = 128
NPAD = 10240
SLICE = NPAD // NSUB
EPAD = NTILE * 80 * CH
TPW = EPAD // (NTILE * CH)
PAD_SRC = N + 8
PAD_DST = N
NB = 8

_mesh = plsc.VectorSubcoreMesh(core_axis_name="c", subcore_axis_name="s")
_sc_params = pltpu.CompilerParams(use_tc_tiling_on_sc=False)


@functools.partial(
    pl.kernel,
    out_type=jax.ShapeDtypeStruct((NCORE, NPAD), jnp.float32),
    mesh=_mesh,
    scratch_types=[
        pltpu.VMEM((TPW, CH), jnp.int32),
        pltpu.VMEM((CH,), jnp.float32),
        pltpu.VMEM((SLICE,), jnp.float32),
        pltpu.VMEM_SHARED((NPAD,), jnp.float32),
        pltpu.SemaphoreType.DMA,
    ],
    compiler_params=_sc_params,
)
def _deg_sc(dst_hbm, out_hbm, idx_v, ones_v, stage_v, deg_sh, sem):
    c = lax.axis_index("c")
    s = lax.axis_index("s")
    w = c * NSUB + s

    idx_dma = pltpu.async_copy(dst_hbm.at[w], idx_v, sem)

    @pl.loop(0, CH // 16)
    def _(i):
        ones_v[pl.ds(i * 16, 16)] = jnp.ones((16,), jnp.float32)

    @pl.loop(0, SLICE // 16)
    def _(i):
        stage_v[pl.ds(i * 16, 16)] = jnp.zeros((16,), jnp.float32)

    pltpu.sync_copy(stage_v, deg_sh.at[pl.ds(s * SLICE, SLICE)])
    plsc.subcore_barrier()
    idx_dma.wait()

    @pl.loop(0, TPW, step=NB)
    def _(j0):
        descs = [pltpu.async_copy(ones_v, deg_sh.at[idx_v.at[j0 + b]], sem,
                                  add=True)
                 for b in range(NB)]
        for d in descs:
            d.wait()

    plsc.subcore_barrier()
    pltpu.sync_copy(deg_sh.at[pl.ds(s * SLICE, SLICE)], stage_v)
    pltpu.sync_copy(stage_v, out_hbm.at[c, pl.ds(s * SLICE, SLICE)])


def _make_prop(W):
    @functools.partial(
        pl.kernel,
        out_type=jax.ShapeDtypeStruct((NCORE, NPAD, W), jnp.float32),
        mesh=_mesh,
        scratch_types=[
            pltpu.VMEM((TPW, CH), jnp.int32),
            pltpu.VMEM((TPW, CH), jnp.int32),
            pltpu.VMEM((NB, CH, W), jnp.float32),
            pltpu.VMEM((SLICE, W), jnp.float32),
            pltpu.VMEM_SHARED((NPAD, W), jnp.float32),
        ] + [pltpu.SemaphoreType.DMA] * (2 * NB),
        compiler_params=_sc_params,
    )
    def _prop(tab_hbm, src_hbm, dst_hbm, out_hbm, si_v, di_v, rows_v,
              stage_v, acc_sh, *sems):
        gsems = sems[:NB]
        ssems = sems[NB:]
        c = lax.axis_index("c")
        s = lax.axis_index("s")
        w = c * NSUB + s

        si_dma = pltpu.async_copy(src_hbm.at[w], si_v, gsems[0])
        di_dma = pltpu.async_copy(dst_hbm.at[w], di_v, ssems[0])

        @pl.loop(0, SLICE)
        def _(i):
            for k in range(W // 16):
                stage_v[i, pl.ds(k * 16, 16)] = jnp.zeros((16,), jnp.float32)

        pltpu.sync_copy(stage_v, acc_sh.at[pl.ds(s * SLICE, SLICE)])
        plsc.subcore_barrier()
        si_dma.wait()
        di_dma.wait()

        @pl.loop(0, TPW, step=NB)
        def _(j0):
            gds = [pltpu.async_copy(tab_hbm.at[si_v.at[j0 + b]],
                                    rows_v.at[b], gsems[b])
                   for b in range(NB)]
            sds = []
            for b in range(NB):
                gds[b].wait()
                sds.append(pltpu.async_copy(rows_v.at[b],
                                            acc_sh.at[di_v.at[j0 + b]],
                                            ssems[b], add=True))
            for d in sds:
                d.wait()

        plsc.subcore_barrier()
        pltpu.sync_copy(acc_sh.at[pl.ds(s * SLICE, SLICE)], stage_v)
        pltpu.sync_copy(stage_v, out_hbm.at[c, pl.ds(s * SLICE, SLICE)])

    return _prop


_prop16 = _make_prop(H1)
_prop32 = _make_prop(H2)


def _mm_body(x_ref, w_ref, o_ref):
    o_ref[...] = jnp.dot(x_ref[...], w_ref[...],
                         preferred_element_type=jnp.float32)


def _scale_body(degt_ref, h_ref, dinv_ref, hs_ref):
    deg = degt_ref[:, 0:1] + degt_ref[:, 1:2] + 1.0
    dinv = lax.rsqrt(deg)
    dinv_ref[...] = dinv
    hs_ref[...] = h_ref[...] * dinv


def _layer_body(a_ref, hs1_ref, dinv_ref, w2_ref, b1_ref, o_ref):
    acc = a_ref[0] + a_ref[1] + hs1_ref[...]
    z1 = jnp.maximum(acc * dinv_ref[...] + b1_ref[...], 0.0)
    o_ref[...] = jnp.dot(z1, w2_ref[...],
                         preferred_element_type=jnp.float32) * dinv_ref[...]


def _final_body(a_ref, hs2_ref, dinv_ref, b2_ref, batch_ref, wo_ref, bo_ref,
                o_ref):
    z2 = jnp.maximum((a_ref[0] + a_ref[1] + hs2_ref[...]) * dinv_ref[...]
                     + b2_ref[...], 0.0)
    gid = lax.broadcasted_iota(jnp.int32, (NUM_GRAPHS, NPAD), 0)
    onehot = (batch_ref[...] == gid).astype(jnp.float32)
    sums = jnp.dot(onehot, z2, preferred_element_type=jnp.float32)
    cnts = jnp.sum(onehot, axis=1, keepdims=True)
    g = sums / jnp.maximum(cnts, 1.0)
    o_ref[...] = jnp.dot(g, wo_ref[...],
                         preferred_element_type=jnp.float32) + bo_ref[...]


def _tc(body, out_shape, *args):
    return pl.pallas_call(body, out_shape=out_shape)(*args)


def kernel(x, edge_index, batch, W1, b1, W2, b2, W_out, b_out):
    f32 = jnp.float32
    src = edge_index[0]
    dst = edge_index[1]
    src_p = jnp.concatenate(
        [src, jnp.full((EPAD - E,), PAD_SRC, jnp.int32)]).reshape(NTILE, TPW, CH)
    dst_p = jnp.concatenate(
        [dst, jnp.full((EPAD - E,), PAD_DST, jnp.int32)]).reshape(NTILE, TPW, CH)
    x_p = jnp.zeros((NPAD, D), f32).at[:N].set(x)
    batch_p = jnp.concatenate(
        [batch, jnp.full((NPAD - N,), NUM_GRAPHS, jnp.int32)]).reshape(1, NPAD)

    deg2 = _deg_sc(dst_p)
    h1 = _tc(_mm_body, jax.ShapeDtypeStruct((NPAD, H1), f32), x_p, W1)
    deg_t = jnp.transpose(deg2)
    dinv, hs1 = _tc(
        _scale_body,
        (jax.ShapeDtypeStruct((NPAD, 1), f32),
         jax.ShapeDtypeStruct((NPAD, H1), f32)),
        deg_t, h1)
    acc1 = _prop16(hs1, src_p, dst_p)
    hs2 = _tc(_layer_body, jax.ShapeDtypeStruct((NPAD, H2), f32),
              acc1, hs1, dinv, W2, b1.reshape(1, H1))
    acc2 = _prop32(hs2, src_p, dst_p)
    out = _tc(_final_body,
              jax.ShapeDtypeStruct((NUM_GRAPHS, NUM_CLASSES), f32),
              acc2, hs2, dinv, b2.reshape(1, H2), batch_p, W_out,
              b_out.reshape(1, NUM_CLASSES))
    return out


# speedup vs baseline: 35.4630x; 1.0085x over previous
import functools

import jax
import jax.numpy as jnp
from jax import lax
from jax.experimental import pallas as pl
from jax.experimental.pallas import tpu as pltpu
from jax.experimental.pallas import tpu_sc as plsc

N = 10000
E = 320000
D = 128
H1 = 16
H2 = 32
NUM_GRAPHS = 16
NUM_CLASSES = 10

NCORE = 2
NSUB = 16
NTILE = NCORE * NSUB
CH = 128
NPAD = 10240
SLICE = NPAD // NSUB
EPAD = NTILE * 80 * CH
TPW = EPAD // (NTILE * CH)
PAD_SRC = N + 8
PAD_DST = N
NB = 8

_mesh = plsc.VectorSubcoreMesh(core_axis_name="c", subcore_axis_name="s")
_sc_params = pltpu.CompilerParams(use_tc_tiling_on_sc=False)


@functools.partial(
    pl.kernel,
    out_type=jax.ShapeDtypeStruct((NCORE, NPAD), jnp.float32),
    mesh=_mesh,
    scratch_types=[
        pltpu.VMEM((TPW, CH), jnp.int32),
        pltpu.VMEM((CH,), jnp.float32),
        pltpu.VMEM((SLICE,), jnp.float32),
        pltpu.VMEM_SHARED((NPAD,), jnp.float32),
        pltpu.SemaphoreType.DMA,
    ],
    compiler_params=_sc_params,
)
def _deg_sc(dst_hbm, out_hbm, idx_v, ones_v, stage_v, deg_sh, sem):
    c = lax.axis_index("c")
    s = lax.axis_index("s")
    w = c * NSUB + s

    idx_dma = pltpu.async_copy(dst_hbm.at[w], idx_v, sem)

    @pl.loop(0, CH // 16)
    def _(i):
        ones_v[pl.ds(i * 16, 16)] = jnp.ones((16,), jnp.float32)

    @pl.loop(0, SLICE // 16)
    def _(i):
        stage_v[pl.ds(i * 16, 16)] = jnp.zeros((16,), jnp.float32)

    pltpu.sync_copy(stage_v, deg_sh.at[pl.ds(s * SLICE, SLICE)])
    plsc.subcore_barrier()
    idx_dma.wait()

    @pl.loop(0, TPW, step=NB)
    def _(j0):
        descs = [pltpu.async_copy(ones_v, deg_sh.at[idx_v.at[j0 + b]], sem,
                                  add=True)
                 for b in range(NB)]
        for d in descs:
            d.wait()

    plsc.subcore_barrier()
    pltpu.sync_copy(deg_sh.at[pl.ds(s * SLICE, SLICE)], stage_v)
    pltpu.sync_copy(stage_v, out_hbm.at[c, pl.ds(s * SLICE, SLICE)])


def _make_prop(W):
    @functools.partial(
        pl.kernel,
        out_type=jax.ShapeDtypeStruct((NCORE, NPAD, W), jnp.float32),
        mesh=_mesh,
        scratch_types=[
            pltpu.VMEM((TPW, CH), jnp.int32),
            pltpu.VMEM((TPW, CH), jnp.int32),
            pltpu.VMEM((NB, CH, W), jnp.float32),
            pltpu.VMEM((SLICE, W), jnp.float32),
            pltpu.VMEM_SHARED((NPAD, W), jnp.float32),
        ] + [pltpu.SemaphoreType.DMA] * (2 * NB),
        compiler_params=_sc_params,
    )
    def _prop(tab_hbm, src_hbm, dst_hbm, out_hbm, si_v, di_v, rows_v,
              stage_v, acc_sh, *sems):
        gsems = sems[:NB]
        ssems = sems[NB:]
        c = lax.axis_index("c")
        s = lax.axis_index("s")
        w = c * NSUB + s

        si_dma = pltpu.async_copy(src_hbm.at[w], si_v, gsems[0])
        di_dma = pltpu.async_copy(dst_hbm.at[w], di_v, ssems[0])

        @pl.loop(0, SLICE)
        def _(i):
            for k in range(W // 16):
                stage_v[i, pl.ds(k * 16, 16)] = jnp.zeros((16,), jnp.float32)

        pltpu.sync_copy(stage_v, acc_sh.at[pl.ds(s * SLICE, SLICE)])
        plsc.subcore_barrier()
        si_dma.wait()
        di_dma.wait()

        @pl.loop(0, TPW, step=NB)
        def _(j0):
            gds = [pltpu.async_copy(tab_hbm.at[si_v.at[j0 + b]],
                                    rows_v.at[b], gsems[b])
                   for b in range(NB)]
            sds = []
            for b in range(NB):
                gds[b].wait()
                sds.append(pltpu.async_copy(rows_v.at[b],
                                            acc_sh.at[di_v.at[j0 + b]],
                                            ssems[b], add=True))
            for d in sds:
                d.wait()

        plsc.subcore_barrier()
        pltpu.sync_copy(acc_sh.at[pl.ds(s * SLICE, SLICE)], stage_v)
        pltpu.sync_copy(stage_v, out_hbm.at[c, pl.ds(s * SLICE, SLICE)])

    return _prop


_prop16 = _make_prop(H1)
_prop32 = _make_prop(H2)


def _mm_body(x_ref, w_ref, o_ref):
    o_ref[...] = jnp.dot(x_ref[...], w_ref[...],
                         preferred_element_type=jnp.float32)


def _scale_body(degt_ref, h_ref, dinv_ref, hs_ref):
    deg = degt_ref[:, 0:1] + degt_ref[:, 1:2] + 1.0
    dinv = lax.rsqrt(deg)
    dinv_ref[...] = dinv
    hs_ref[...] = h_ref[...] * dinv


def _layer_body(a_ref, hs1_ref, dinv_ref, w2_ref, b1_ref, o_ref):
    acc = a_ref[0] + a_ref[1] + hs1_ref[...]
    z1 = jnp.maximum(acc * dinv_ref[...] + b1_ref[...], 0.0)
    o_ref[...] = jnp.dot(z1, w2_ref[...],
                         preferred_element_type=jnp.float32) * dinv_ref[...]


def _final_body(a_ref, hs2_ref, dinv_ref, b2_ref, batch_ref, wo_ref, bo_ref,
                o_ref):
    z2 = jnp.maximum((a_ref[0] + a_ref[1] + hs2_ref[...]) * dinv_ref[...]
                     + b2_ref[...], 0.0)
    gid = lax.broadcasted_iota(jnp.int32, (NUM_GRAPHS, NPAD), 0)
    onehot = (batch_ref[...] == gid).astype(jnp.float32)
    sums = jnp.dot(onehot, z2, preferred_element_type=jnp.float32)
    cnts = jnp.sum(onehot, axis=1, keepdims=True)
    g = sums / jnp.maximum(cnts, 1.0)
    o_ref[...] = jnp.dot(g, wo_ref[...],
                         preferred_element_type=jnp.float32) + bo_ref[...]


def _tc(body, out_shape, *args):
    return pl.pallas_call(body, out_shape=out_shape)(*args)


def kernel(x, edge_index, batch, W1, b1, W2, b2, W_out, b_out):
    f32 = jnp.float32
    src = edge_index[0]
    dst = edge_index[1]
    pad_dst = PAD_DST + jnp.arange(EPAD - E, dtype=jnp.int32) % (NPAD - N)
    src_p = jnp.concatenate(
        [src, jnp.full((EPAD - E,), PAD_SRC, jnp.int32)]).reshape(NTILE, TPW, CH)
    dst_p = jnp.concatenate([dst, pad_dst]).reshape(NTILE, TPW, CH)
    x_p = jnp.zeros((NPAD, D), f32).at[:N].set(x)
    batch_p = jnp.concatenate(
        [batch, jnp.full((NPAD - N,), NUM_GRAPHS, jnp.int32)]).reshape(1, NPAD)

    deg2 = _deg_sc(dst_p)
    h1 = _tc(_mm_body, jax.ShapeDtypeStruct((NPAD, H1), f32), x_p, W1)
    deg_t = jnp.transpose(deg2)
    dinv, hs1 = _tc(
        _scale_body,
        (jax.ShapeDtypeStruct((NPAD, 1), f32),
         jax.ShapeDtypeStruct((NPAD, H1), f32)),
        deg_t, h1)
    acc1 = _prop16(hs1, src_p, dst_p)
    hs2 = _tc(_layer_body, jax.ShapeDtypeStruct((NPAD, H2), f32),
              acc1, hs1, dinv, W2, b1.reshape(1, H1))
    acc2 = _prop32(hs2, src_p, dst_p)
    out = _tc(_final_body,
              jax.ShapeDtypeStruct((NUM_GRAPHS, NUM_CLASSES), f32),
              acc2, hs2, dinv, b2.reshape(1, H2), batch_p, W_out,
              b_out.reshape(1, NUM_CLASSES))
    return out


# speedup vs baseline: 58.9285x; 1.6617x over previous
import functools

import jax
import jax.numpy as jnp
from jax import lax
from jax.experimental import pallas as pl
from jax.experimental.pallas import tpu as pltpu
from jax.experimental.pallas import tpu_sc as plsc

N = 10000
E = 320000
D = 128
H1 = 16
H2 = 32
NUM_GRAPHS = 16
NUM_CLASSES = 10

NCORE = 2
NSUB = 16
NTILE = NCORE * NSUB
CH = 128
NPAD = 10240
SLICE = NPAD // NSUB
EPAD = NTILE * 80 * CH
TPW = EPAD // (NTILE * CH)
PAD_SRC = N + 8
PAD_DST = N
NB = 8

_mesh = plsc.VectorSubcoreMesh(core_axis_name="c", subcore_axis_name="s")
_sc_params = pltpu.CompilerParams(use_tc_tiling_on_sc=False)


@functools.partial(
    pl.kernel,
    out_type=jax.ShapeDtypeStruct((NCORE, NPAD), jnp.float32),
    mesh=_mesh,
    scratch_types=[
        pltpu.VMEM((TPW, CH), jnp.int32),
        pltpu.VMEM((CH,), jnp.float32),
        pltpu.VMEM((SLICE,), jnp.float32),
        pltpu.VMEM_SHARED((NPAD,), jnp.float32),
        pltpu.SemaphoreType.DMA,
    ],
    compiler_params=_sc_params,
)
def _deg_sc(dst_hbm, out_hbm, idx_v, ones_v, stage_v, deg_sh, sem):
    c = lax.axis_index("c")
    s = lax.axis_index("s")
    w = c * NSUB + s

    idx_dma = pltpu.async_copy(dst_hbm.at[w], idx_v, sem)

    @pl.loop(0, CH // 16)
    def _(i):
        ones_v[pl.ds(i * 16, 16)] = jnp.ones((16,), jnp.float32)

    @pl.loop(0, SLICE // 16)
    def _(i):
        stage_v[pl.ds(i * 16, 16)] = jnp.zeros((16,), jnp.float32)

    pltpu.sync_copy(stage_v, deg_sh.at[pl.ds(s * SLICE, SLICE)])
    plsc.subcore_barrier()
    idx_dma.wait()

    @pl.loop(0, TPW, step=NB)
    def _(j0):
        descs = [pltpu.async_copy(ones_v, deg_sh.at[idx_v.at[j0 + b]], sem,
                                  add=True)
                 for b in range(NB)]
        for d in descs:
            d.wait()

    plsc.subcore_barrier()
    pltpu.sync_copy(deg_sh.at[pl.ds(s * SLICE, SLICE)], stage_v)
    pltpu.sync_copy(stage_v, out_hbm.at[c, pl.ds(s * SLICE, SLICE)])


def _make_prop(W):
    @functools.partial(
        pl.kernel,
        out_type=jax.ShapeDtypeStruct((NCORE, NPAD, W), jnp.float32),
        mesh=_mesh,
        scratch_types=[
            pltpu.VMEM((TPW, CH), jnp.int32),
            pltpu.VMEM((TPW, CH), jnp.int32),
            pltpu.VMEM((NB, CH, W), jnp.float32),
            pltpu.VMEM((SLICE, W), jnp.float32),
            pltpu.VMEM_SHARED((NPAD, W), jnp.float32),
        ] + [pltpu.SemaphoreType.DMA] * (2 * NB),
        compiler_params=_sc_params,
    )
    def _prop(tab_hbm, src_hbm, dst_hbm, out_hbm, si_v, di_v, rows_v,
              stage_v, acc_sh, *sems):
        gsems = sems[:NB]
        ssems = sems[NB:]
        c = lax.axis_index("c")
        s = lax.axis_index("s")
        w = c * NSUB + s

        si_dma = pltpu.async_copy(src_hbm.at[w], si_v, gsems[0])
        di_dma = pltpu.async_copy(dst_hbm.at[w], di_v, ssems[0])

        @pl.loop(0, SLICE)
        def _(i):
            for k in range(W // 16):
                stage_v[i, pl.ds(k * 16, 16)] = jnp.zeros((16,), jnp.float32)

        pltpu.sync_copy(stage_v, acc_sh.at[pl.ds(s * SLICE, SLICE)])
        plsc.subcore_barrier()
        si_dma.wait()
        di_dma.wait()

        @pl.loop(0, TPW, step=NB)
        def _(j0):
            gds = [pltpu.async_copy(tab_hbm.at[si_v.at[j0 + b]],
                                    rows_v.at[b], gsems[b])
                   for b in range(NB)]
            sds = []
            for b in range(NB):
                gds[b].wait()
                sds.append(pltpu.async_copy(rows_v.at[b],
                                            acc_sh.at[di_v.at[j0 + b]],
                                            ssems[b], add=True))
            for d in sds:
                d.wait()

        plsc.subcore_barrier()
        pltpu.sync_copy(acc_sh.at[pl.ds(s * SLICE, SLICE)], stage_v)
        pltpu.sync_copy(stage_v, out_hbm.at[c, pl.ds(s * SLICE, SLICE)])

    return _prop


_prop16 = _make_prop(H1)
_prop32 = _make_prop(H2)


def _mm_body(x_ref, w_ref, o_ref):
    o_ref[...] = jnp.dot(x_ref[...], w_ref[...],
                         preferred_element_type=jnp.float32)


def _scale_body(degt_ref, h_ref, dinv_ref, hs_ref):
    deg = degt_ref[:, 0:1] + degt_ref[:, 1:2] + 1.0
    rid = lax.broadcasted_iota(jnp.int32, (NPAD, 1), 0)
    dinv = jnp.where(rid < N, lax.rsqrt(deg), 0.0)
    dinv_ref[...] = dinv
    hs_ref[...] = h_ref[...] * dinv


def _layer_body(a_ref, hs1_ref, dinv_ref, w2_ref, b1_ref, o_ref):
    acc = a_ref[0] + a_ref[1] + hs1_ref[...]
    z1 = jnp.maximum(acc * dinv_ref[...] + b1_ref[...], 0.0)
    o_ref[...] = jnp.dot(z1, w2_ref[...],
                         preferred_element_type=jnp.float32) * dinv_ref[...]


def _final_body(a_ref, hs2_ref, dinv_ref, b2_ref, batch_ref, wo_ref, bo_ref,
                o_ref):
    z2 = jnp.maximum((a_ref[0] + a_ref[1] + hs2_ref[...]) * dinv_ref[...]
                     + b2_ref[...], 0.0)
    gid = lax.broadcasted_iota(jnp.int32, (NUM_GRAPHS, NPAD), 0)
    onehot = (batch_ref[...] == gid).astype(jnp.float32)
    sums = jnp.dot(onehot, z2, preferred_element_type=jnp.float32)
    cnts = jnp.sum(onehot, axis=1, keepdims=True)
    g = sums / jnp.maximum(cnts, 1.0)
    o_ref[...] = jnp.dot(g, wo_ref[...],
                         preferred_element_type=jnp.float32) + bo_ref[...]


def _tc(body, out_shape, *args):
    return pl.pallas_call(body, out_shape=out_shape)(*args)


def kernel(x, edge_index, batch, W1, b1, W2, b2, W_out, b_out):
    f32 = jnp.float32
    src = edge_index[0]
    dst = edge_index[1]
    pad_cyc = jnp.arange(EPAD - E, dtype=jnp.int32) % (NPAD - N)
    src_p = jnp.concatenate([src, N + pad_cyc]).reshape(NTILE, TPW, CH)
    dst_p = jnp.concatenate([dst, N + pad_cyc]).reshape(NTILE, TPW, CH)
    x_p = jnp.zeros((NPAD, D), f32).at[:N].set(x)
    batch_p = jnp.concatenate(
        [batch, jnp.full((NPAD - N,), NUM_GRAPHS, jnp.int32)]).reshape(1, NPAD)

    deg2 = _deg_sc(dst_p)
    h1 = _tc(_mm_body, jax.ShapeDtypeStruct((NPAD, H1), f32), x_p, W1)
    deg_t = jnp.transpose(deg2)
    dinv, hs1 = _tc(
        _scale_body,
        (jax.ShapeDtypeStruct((NPAD, 1), f32),
         jax.ShapeDtypeStruct((NPAD, H1), f32)),
        deg_t, h1)
    acc1 = _prop16(hs1, src_p, dst_p)
    hs2 = _tc(_layer_body, jax.ShapeDtypeStruct((NPAD, H2), f32),
              acc1, hs1, dinv, W2, b1.reshape(1, H1))
    acc2 = _prop32(hs2, src_p, dst_p)
    out = _tc(_final_body,
              jax.ShapeDtypeStruct((NUM_GRAPHS, NUM_CLASSES), f32),
              acc2, hs2, dinv, b2.reshape(1, H2), batch_p, W_out,
              b_out.reshape(1, NUM_CLASSES))
    return out


# speedup vs baseline: 73.6093x; 1.2491x over previous
import functools

import jax
import jax.numpy as jnp
from jax import lax
from jax.experimental import pallas as pl
from jax.experimental.pallas import tpu as pltpu
from jax.experimental.pallas import tpu_sc as plsc

N = 10000
E = 320000
D = 128
H1 = 16
H2 = 32
NUM_GRAPHS = 16
NUM_CLASSES = 10

NCORE = 2
NSUB = 16
NTILE = NCORE * NSUB
CH = 80
TPW = 125
NB = 5
NPAD = 10240
SLICE = NPAD // NSUB

_mesh = plsc.VectorSubcoreMesh(core_axis_name="c", subcore_axis_name="s")
_sc_params = pltpu.CompilerParams(use_tc_tiling_on_sc=False)


@functools.partial(
    pl.kernel,
    out_type=jax.ShapeDtypeStruct((NCORE, NPAD), jnp.float32),
    mesh=_mesh,
    scratch_types=[
        pltpu.VMEM((TPW, CH), jnp.int32),
        pltpu.VMEM((CH,), jnp.float32),
        pltpu.VMEM((SLICE,), jnp.float32),
        pltpu.VMEM_SHARED((NPAD,), jnp.float32),
        pltpu.SemaphoreType.DMA,
    ],
    compiler_params=_sc_params,
)
def _deg_sc(dst_hbm, out_hbm, idx_v, ones_v, stage_v, deg_sh, sem):
    c = lax.axis_index("c")
    s = lax.axis_index("s")
    w = c * NSUB + s

    idx_dma = pltpu.async_copy(dst_hbm.at[w], idx_v, sem)

    @pl.loop(0, CH // 16)
    def _(i):
        ones_v[pl.ds(i * 16, 16)] = jnp.ones((16,), jnp.float32)

    @pl.loop(0, SLICE // 16)
    def _(i):
        stage_v[pl.ds(i * 16, 16)] = jnp.zeros((16,), jnp.float32)

    pltpu.sync_copy(stage_v, deg_sh.at[pl.ds(s * SLICE, SLICE)])
    plsc.subcore_barrier()
    idx_dma.wait()

    @pl.loop(0, TPW, step=NB)
    def _(j0):
        descs = [pltpu.async_copy(ones_v, deg_sh.at[idx_v.at[j0 + b]], sem,
                                  add=True)
                 for b in range(NB)]
        for d in descs:
            d.wait()

    plsc.subcore_barrier()
    pltpu.sync_copy(deg_sh.at[pl.ds(s * SLICE, SLICE)], stage_v)
    pltpu.sync_copy(stage_v, out_hbm.at[c, pl.ds(s * SLICE, SLICE)])


def _make_prop(W):
    @functools.partial(
        pl.kernel,
        out_type=jax.ShapeDtypeStruct((NCORE, NPAD, W), jnp.float32),
        mesh=_mesh,
        scratch_types=[
            pltpu.VMEM((TPW, CH), jnp.int32),
            pltpu.VMEM((TPW, CH), jnp.int32),
            pltpu.VMEM((NB, CH, W), jnp.float32),
            pltpu.VMEM((SLICE, W), jnp.float32),
            pltpu.VMEM_SHARED((NPAD, W), jnp.float32),
        ] + [pltpu.SemaphoreType.DMA] * (2 * NB),
        compiler_params=_sc_params,
    )
    def _prop(tab_hbm, src_hbm, dst_hbm, out_hbm, si_v, di_v, rows_v,
              stage_v, acc_sh, *sems):
        gsems = sems[:NB]
        ssems = sems[NB:]
        c = lax.axis_index("c")
        s = lax.axis_index("s")
        w = c * NSUB + s

        si_dma = pltpu.async_copy(src_hbm.at[w], si_v, gsems[0])
        di_dma = pltpu.async_copy(dst_hbm.at[w], di_v, ssems[0])

        @pl.loop(0, SLICE)
        def _(i):
            for k in range(W // 16):
                stage_v[i, pl.ds(k * 16, 16)] = jnp.zeros((16,), jnp.float32)

        pltpu.sync_copy(stage_v, acc_sh.at[pl.ds(s * SLICE, SLICE)])
        plsc.subcore_barrier()
        si_dma.wait()
        di_dma.wait()

        @pl.loop(0, TPW, step=NB)
        def _(j0):
            gds = [pltpu.async_copy(tab_hbm.at[si_v.at[j0 + b]],
                                    rows_v.at[b], gsems[b])
                   for b in range(NB)]
            sds = []
            for b in range(NB):
                gds[b].wait()
                sds.append(pltpu.async_copy(rows_v.at[b],
                                            acc_sh.at[di_v.at[j0 + b]],
                                            ssems[b], add=True))
            for d in sds:
                d.wait()

        plsc.subcore_barrier()
        pltpu.sync_copy(acc_sh.at[pl.ds(s * SLICE, SLICE)], stage_v)
        pltpu.sync_copy(stage_v, out_hbm.at[c, pl.ds(s * SLICE, SLICE)])

    return _prop


_prop16 = _make_prop(H1)
_prop32 = _make_prop(H2)


R16 = N * H1 // 128
R32 = N * H2 // 128


def _expand_dinv(dinv_l, rep, rows):
    width = 128 * rep
    l_i = lax.broadcasted_iota(jnp.int32, (128, width), 0)
    c_i = lax.broadcasted_iota(jnp.int32, (128, width), 1)
    m = (l_i == c_i // rep).astype(jnp.float32)
    e = jnp.dot(dinv_l, m, preferred_element_type=jnp.float32)
    return jnp.reshape(e, (NPAD * rep // 128, 128))[:rows]


def _pre_body(deg_ref, x_ref, w1_ref, hs1_ref, de16_ref, de32_ref):
    deg = deg_ref[0] + deg_ref[1] + 1.0
    dinv_l = lax.rsqrt(deg)
    e16 = _expand_dinv(dinv_l, H1, R16)
    e32 = _expand_dinv(dinv_l, H2, R32)
    de16_ref[...] = e16
    de32_ref[...] = e32
    x8 = jnp.reshape(x_ref[...], (R16, 8 * D))
    w1r = jnp.reshape(jnp.broadcast_to(w1_ref[...][:, None, :],
                                       (D, 8, H1)), (D, 128))
    w1big = jnp.concatenate([w1r] * 8, axis=0)
    r_i = lax.broadcasted_iota(jnp.int32, (8 * D, 128), 0)
    c_i = lax.broadcasted_iota(jnp.int32, (8 * D, 128), 1)
    w1big = jnp.where(r_i // D == c_i // H1, w1big, 0.0)
    h_p = jnp.dot(x8, w1big, preferred_element_type=jnp.float32)
    hs1_ref[...] = h_p * e16


def _layer_body(a_ref, hs1_ref, de16_ref, de32_ref, w2_ref, b1_ref, o_ref):
    accp = a_ref[0, :R16] + a_ref[1, :R16] + hs1_ref[...]
    b1t = jnp.reshape(jnp.broadcast_to(b1_ref[...][:, None, :],
                                       (1, 128 // H1, H1)), (1, 128))
    z1p = jnp.maximum(accp * de16_ref[...] + b1t, 0.0)
    w2r = jnp.reshape(jnp.broadcast_to(w2_ref[...][:, None, :],
                                       (H1, 256 // H2, H2)), (H1, 256))
    w2big = jnp.concatenate([w2r] * (128 // H1), axis=0)
    r_i = lax.broadcasted_iota(jnp.int32, (128, 256), 0)
    c_i = lax.broadcasted_iota(jnp.int32, (128, 256), 1)
    w2big = jnp.where(r_i // H1 == c_i // H2, w2big, 0.0)
    h2 = jnp.dot(z1p, w2big, preferred_element_type=jnp.float32)
    h2p = jnp.reshape(h2, (R32, 128))
    o_ref[...] = h2p * de32_ref[...]


def _final_body(a_ref, hs2_ref, de32_ref, b2_ref, batch_ref, wo_ref,
                bo_ref, o_ref):
    accp = a_ref[0, :R32] + a_ref[1, :R32] + hs2_ref[...]
    b2t = jnp.reshape(jnp.broadcast_to(b2_ref[...][:, None, :],
                                       (1, 128 // H2, H2)), (1, 128))
    z2p = jnp.maximum(accp * de32_ref[...] + b2t, 0.0)
    gid = lax.broadcasted_iota(jnp.int32, (NUM_GRAPHS, R32), 0)
    sums = jnp.zeros((NUM_GRAPHS, H2), jnp.float32)
    cnts = jnp.zeros((NUM_GRAPHS, 1), jnp.float32)
    for b in range(4):
        pb = (batch_ref[b:b + 1, :] == gid).astype(jnp.float32)
        sb = jnp.dot(pb, z2p, preferred_element_type=jnp.float32)
        sums = sums + sb[:, b * H2:(b + 1) * H2]
        cnts = cnts + jnp.sum(pb, axis=1, keepdims=True)
    g = sums / jnp.maximum(cnts, 1.0)
    o_ref[...] = jnp.dot(g, wo_ref[...],
                         preferred_element_type=jnp.float32) + bo_ref[...]


def _tc(body, out_shape, *args):
    return pl.pallas_call(body, out_shape=out_shape)(*args)


def kernel(x, edge_index, batch, W1, b1, W2, b2, W_out, b_out):
    f32 = jnp.float32
    sd = jax.ShapeDtypeStruct
    src_p = edge_index[0].reshape(NTILE, TPW, CH)
    dst_p = edge_index[1].reshape(NTILE, TPW, CH)
    batch_s = jnp.transpose(batch.reshape(R32, 4))

    deg2 = _deg_sc(dst_p)
    hs1_lin, de16, de32 = _tc(
        _pre_body,
        (sd((R16, 128), f32), sd((R16, 128), f32), sd((R32, 128), f32)),
        deg2.reshape(NCORE, NPAD // 128, 128), x, W1)
    acc1 = _prop16(hs1_lin.reshape(N, H1), src_p, dst_p)
    hs2_lin = _tc(
        _layer_body, sd((R32, 128), f32),
        acc1.reshape(NCORE, NPAD * H1 // 128, 128), hs1_lin, de16, de32,
        W2, b1.reshape(1, H1))
    acc2 = _prop32(hs2_lin.reshape(N, H2), src_p, dst_p)
    out = _tc(
        _final_body, sd((NUM_GRAPHS, NUM_CLASSES), f32),
        acc2.reshape(NCORE, NPAD * H2 // 128, 128), hs2_lin, de32,
        b2.reshape(1, H2), batch_s, W_out,
        b_out.reshape(1, NUM_CLASSES))
    return out


# speedup vs baseline: 89.4807x; 1.2156x over previous
import functools

import jax
import jax.numpy as jnp
from jax import lax
from jax.experimental import pallas as pl
from jax.experimental.pallas import tpu as pltpu
from jax.experimental.pallas import tpu_sc as plsc

N = 10000
E = 320000
D = 128
H1 = 16
H2 = 32
NUM_GRAPHS = 16
NUM_CLASSES = 10

NCORE = 2
NSUB = 16
NTILE = NCORE * NSUB
CH = 125
TPW = 80
NB = 8
NPAD = 10240
SLICE = NPAD // NSUB

_mesh = plsc.VectorSubcoreMesh(core_axis_name="c", subcore_axis_name="s")
_sc_params = pltpu.CompilerParams(use_tc_tiling_on_sc=False)


@functools.partial(
    pl.kernel,
    out_type=jax.ShapeDtypeStruct((NCORE, NPAD), jnp.float32),
    mesh=_mesh,
    scratch_types=[
        pltpu.VMEM((TPW, CH), jnp.int32),
        pltpu.VMEM((128,), jnp.float32),
        pltpu.VMEM((SLICE,), jnp.float32),
        pltpu.VMEM_SHARED((NPAD,), jnp.float32),
        pltpu.SemaphoreType.DMA,
    ],
    compiler_params=_sc_params,
)
def _deg_sc(dst_hbm, out_hbm, idx_v, ones_v, stage_v, deg_sh, sem):
    c = lax.axis_index("c")
    s = lax.axis_index("s")
    w = c * NSUB + s

    idx_dma = pltpu.async_copy(dst_hbm.at[w], idx_v, sem)

    @pl.loop(0, 128 // 16)
    def _(i):
        ones_v[pl.ds(i * 16, 16)] = jnp.ones((16,), jnp.float32)

    @pl.loop(0, SLICE // 16)
    def _(i):
        stage_v[pl.ds(i * 16, 16)] = jnp.zeros((16,), jnp.float32)

    pltpu.sync_copy(stage_v, deg_sh.at[pl.ds(s * SLICE, SLICE)])
    plsc.subcore_barrier()
    idx_dma.wait()

    @pl.loop(0, TPW, step=NB)
    def _(j0):
        descs = [pltpu.async_copy(ones_v.at[pl.ds(0, CH)],
                                  deg_sh.at[idx_v.at[j0 + b]], sem,
                                  add=True)
                 for b in range(NB)]
        for d in descs:
            d.wait()

    plsc.subcore_barrier()
    pltpu.sync_copy(deg_sh.at[pl.ds(s * SLICE, SLICE)], stage_v)
    pltpu.sync_copy(stage_v, out_hbm.at[c, pl.ds(s * SLICE, SLICE)])


def _make_prop(W):
    @functools.partial(
        pl.kernel,
        out_type=jax.ShapeDtypeStruct((NCORE, NPAD, W), jnp.float32),
        mesh=_mesh,
        scratch_types=[
            pltpu.VMEM((TPW, CH), jnp.int32),
            pltpu.VMEM((TPW, CH), jnp.int32),
            pltpu.VMEM((NB, CH, W), jnp.float32),
            pltpu.VMEM((SLICE, W), jnp.float32),
            pltpu.VMEM_SHARED((NPAD, W), jnp.float32),
        ] + [pltpu.SemaphoreType.DMA] * (2 * NB),
        compiler_params=_sc_params,
    )
    def _prop(tab_hbm, src_hbm, dst_hbm, out_hbm, si_v, di_v, rows_v,
              stage_v, acc_sh, *sems):
        gsems = sems[:NB]
        ssems = sems[NB:]
        c = lax.axis_index("c")
        s = lax.axis_index("s")
        w = c * NSUB + s

        si_dma = pltpu.async_copy(src_hbm.at[w], si_v, gsems[0])
        di_dma = pltpu.async_copy(dst_hbm.at[w], di_v, ssems[0])

        @pl.loop(0, SLICE)
        def _(i):
            for k in range(W // 16):
                stage_v[i, pl.ds(k * 16, 16)] = jnp.zeros((16,), jnp.float32)

        pltpu.sync_copy(stage_v, acc_sh.at[pl.ds(s * SLICE, SLICE)])
        plsc.subcore_barrier()
        si_dma.wait()
        di_dma.wait()

        @pl.loop(0, TPW, step=NB)
        def _(j0):
            gds = [pltpu.async_copy(tab_hbm.at[si_v.at[j0 + b]],
                                    rows_v.at[b], gsems[b])
                   for b in range(NB)]
            sds = []
            for b in range(NB):
                gds[b].wait()
                sds.append(pltpu.async_copy(rows_v.at[b],
                                            acc_sh.at[di_v.at[j0 + b]],
                                            ssems[b], add=True))
            for d in sds:
                d.wait()

        plsc.subcore_barrier()
        pltpu.sync_copy(acc_sh.at[pl.ds(s * SLICE, SLICE)], stage_v)
        pltpu.sync_copy(stage_v, out_hbm.at[c, pl.ds(s * SLICE, SLICE)])

    return _prop


_prop16 = _make_prop(H1)
_prop32 = _make_prop(H2)


R16 = N * H1 // 128
R32 = N * H2 // 128


def _expand_dinv(dinv_l, rep, rows):
    width = 128 * rep
    l_i = lax.broadcasted_iota(jnp.int32, (128, width), 0)
    c_i = lax.broadcasted_iota(jnp.int32, (128, width), 1)
    m = (l_i == c_i // rep).astype(jnp.float32)
    e = jnp.dot(dinv_l, m, preferred_element_type=jnp.float32)
    return jnp.reshape(e, (NPAD * rep // 128, 128))[:rows]


def _edges_body(ei_ref, s_ref, d_ref):
    s_ref[...] = jnp.reshape(ei_ref[0:1, :], (E // 128, 128))
    d_ref[...] = jnp.reshape(ei_ref[1:2, :], (E // 128, 128))


def _pre_body(deg_ref, x_ref, w1_ref, hs1_ref, de16_ref, de32_ref):
    deg = deg_ref[0] + deg_ref[1] + 1.0
    dinv_l = lax.rsqrt(deg)
    e16 = _expand_dinv(dinv_l, H1, R16)
    e32 = _expand_dinv(dinv_l, H2, R32)
    de16_ref[...] = e16
    de32_ref[...] = e32
    x8 = jnp.reshape(x_ref[...], (R16, 8 * D))
    w1r = jnp.reshape(jnp.broadcast_to(w1_ref[...][:, None, :],
                                       (D, 8, H1)), (D, 128))
    w1big = jnp.concatenate([w1r] * 8, axis=0)
    r_i = lax.broadcasted_iota(jnp.int32, (8 * D, 128), 0)
    c_i = lax.broadcasted_iota(jnp.int32, (8 * D, 128), 1)
    w1big = jnp.where(r_i // D == c_i // H1, w1big, 0.0)
    h_p = jnp.dot(x8, w1big, preferred_element_type=jnp.float32)
    hs1_ref[...] = h_p * e16


def _layer_body(a_ref, hs1_ref, de16_ref, de32_ref, w2_ref, b1_ref, o_ref):
    accp = a_ref[0, :R16] + a_ref[1, :R16] + hs1_ref[...]
    b1t = jnp.reshape(jnp.broadcast_to(b1_ref[...][:, None, :],
                                       (1, 128 // H1, H1)), (1, 128))
    z1p = jnp.maximum(accp * de16_ref[...] + b1t, 0.0)
    w2r = jnp.reshape(jnp.broadcast_to(w2_ref[...][:, None, :],
                                       (H1, 256 // H2, H2)), (H1, 256))
    w2big = jnp.concatenate([w2r] * (128 // H1), axis=0)
    r_i = lax.broadcasted_iota(jnp.int32, (128, 256), 0)
    c_i = lax.broadcasted_iota(jnp.int32, (128, 256), 1)
    w2big = jnp.where(r_i // H1 == c_i // H2, w2big, 0.0)
    h2 = jnp.dot(z1p, w2big, preferred_element_type=jnp.float32)
    h2p = jnp.reshape(h2, (R32, 128))
    o_ref[...] = h2p * de32_ref[...]


def _final_body(a_ref, hs2_ref, de32_ref, b2_ref, batch_ref, wo_ref,
                bo_ref, o_ref):
    accp = a_ref[0, :R32] + a_ref[1, :R32] + hs2_ref[...]
    b2t = jnp.reshape(jnp.broadcast_to(b2_ref[...][:, None, :],
                                       (1, 128 // H2, H2)), (1, 128))
    z2p = jnp.maximum(accp * de32_ref[...] + b2t, 0.0)
    gid = lax.broadcasted_iota(jnp.int32, (NUM_GRAPHS, R32), 0)
    sums = jnp.zeros((NUM_GRAPHS, H2), jnp.float32)
    cnts = jnp.zeros((NUM_GRAPHS, 1), jnp.float32)
    for b in range(4):
        pb = (batch_ref[b:b + 1, :] == gid).astype(jnp.float32)
        sb = jnp.dot(pb, z2p, preferred_element_type=jnp.float32)
        sums = sums + sb[:, b * H2:(b + 1) * H2]
        cnts = cnts + jnp.sum(pb, axis=1, keepdims=True)
    g = sums / jnp.maximum(cnts, 1.0)
    o_ref[...] = jnp.dot(g, wo_ref[...],
                         preferred_element_type=jnp.float32) + bo_ref[...]


def _tc(body, out_shape, *args):
    return pl.pallas_call(body, out_shape=out_shape)(*args)


def kernel(x, edge_index, batch, W1, b1, W2, b2, W_out, b_out):
    f32 = jnp.float32
    sd = jax.ShapeDtypeStruct
    srcl, dstl = _tc(
        _edges_body,
        (sd((E // 128, 128), jnp.int32), sd((E // 128, 128), jnp.int32)),
        edge_index)
    src_p = srcl.reshape(NTILE, TPW, CH)
    dst_p = dstl.reshape(NTILE, TPW, CH)
    batch_s = jnp.transpose(batch.reshape(R32, 4))

    deg2 = _deg_sc(dst_p)
    hs1_lin, de16, de32 = _tc(
        _pre_body,
        (sd((R16, 128), f32), sd((R16, 128), f32), sd((R32, 128), f32)),
        deg2.reshape(NCORE, NPAD // 128, 128), x, W1)
    acc1 = _prop16(hs1_lin.reshape(N, H1), src_p, dst_p)
    hs2_lin = _tc(
        _layer_body, sd((R32, 128), f32),
        acc1.reshape(NCORE, NPAD * H1 // 128, 128), hs1_lin, de16, de32,
        W2, b1.reshape(1, H1))
    acc2 = _prop32(hs2_lin.reshape(N, H2), src_p, dst_p)
    out = _tc(
        _final_body, sd((NUM_GRAPHS, NUM_CLASSES), f32),
        acc2.reshape(NCORE, NPAD * H2 // 128, 128), hs2_lin, de32,
        b2.reshape(1, H2), batch_s, W_out,
        b_out.reshape(1, NUM_CLASSES))
    return out


# speedup vs baseline: 95.3513x; 1.0656x over previous
import functools

import jax
import jax.numpy as jnp
from jax import lax
from jax.experimental import pallas as pl
from jax.experimental.pallas import tpu as pltpu
from jax.experimental.pallas import tpu_sc as plsc

N = 10000
E = 320000
D = 128
H1 = 16
H2 = 32
NUM_GRAPHS = 16
NUM_CLASSES = 10

NCORE = 2
NSUB = 16
NTILE = NCORE * NSUB
CH = 128
TPW = 80
NB = 8
EPAD = NTILE * TPW * CH
ER = E // 128
EPR = EPAD // 128
NPAD = 10240
SLICE = NPAD // NSUB

_mesh = plsc.VectorSubcoreMesh(core_axis_name="c", subcore_axis_name="s")
_sc_params = pltpu.CompilerParams(use_tc_tiling_on_sc=False)


@functools.partial(
    pl.kernel,
    out_type=jax.ShapeDtypeStruct((NCORE, NPAD), jnp.float32),
    mesh=_mesh,
    scratch_types=[
        pltpu.VMEM((TPW, CH), jnp.int32),
        pltpu.VMEM((128,), jnp.float32),
        pltpu.VMEM((SLICE,), jnp.float32),
        pltpu.VMEM_SHARED((NPAD,), jnp.float32),
        pltpu.SemaphoreType.DMA,
    ],
    compiler_params=_sc_params,
)
def _deg_sc(dst_hbm, out_hbm, idx_v, ones_v, stage_v, deg_sh, sem):
    c = lax.axis_index("c")
    s = lax.axis_index("s")
    w = c * NSUB + s

    idx_dma = pltpu.async_copy(dst_hbm.at[w], idx_v, sem)

    @pl.loop(0, 128 // 16)
    def _(i):
        ones_v[pl.ds(i * 16, 16)] = jnp.ones((16,), jnp.float32)

    @pl.loop(0, SLICE // 16)
    def _(i):
        stage_v[pl.ds(i * 16, 16)] = jnp.zeros((16,), jnp.float32)

    pltpu.sync_copy(stage_v, deg_sh.at[pl.ds(s * SLICE, SLICE)])
    plsc.subcore_barrier()
    idx_dma.wait()

    @pl.loop(0, TPW, step=NB)
    def _(j0):
        descs = [pltpu.async_copy(ones_v.at[pl.ds(0, CH)],
                                  deg_sh.at[idx_v.at[j0 + b]], sem,
                                  add=True)
                 for b in range(NB)]
        for d in descs:
            d.wait()

    plsc.subcore_barrier()
    pltpu.sync_copy(deg_sh.at[pl.ds(s * SLICE, SLICE)], stage_v)
    pltpu.sync_copy(stage_v, out_hbm.at[c, pl.ds(s * SLICE, SLICE)])


def _make_prop(W):
    @functools.partial(
        pl.kernel,
        out_type=jax.ShapeDtypeStruct((NCORE, NPAD, W), jnp.float32),
        mesh=_mesh,
        scratch_types=[
            pltpu.VMEM((TPW, CH), jnp.int32),
            pltpu.VMEM((TPW, CH), jnp.int32),
            pltpu.VMEM((NB, CH, W), jnp.float32),
            pltpu.VMEM((SLICE, W), jnp.float32),
            pltpu.VMEM_SHARED((NPAD, W), jnp.float32),
        ] + [pltpu.SemaphoreType.DMA] * (2 * NB),
        compiler_params=_sc_params,
    )
    def _prop(tab_hbm, src_hbm, dst_hbm, out_hbm, si_v, di_v, rows_v,
              stage_v, acc_sh, *sems):
        gsems = sems[:NB]
        ssems = sems[NB:]
        c = lax.axis_index("c")
        s = lax.axis_index("s")
        w = c * NSUB + s

        si_dma = pltpu.async_copy(src_hbm.at[w], si_v, gsems[0])
        di_dma = pltpu.async_copy(dst_hbm.at[w], di_v, ssems[0])

        @pl.loop(0, SLICE)
        def _(i):
            for k in range(W // 16):
                stage_v[i, pl.ds(k * 16, 16)] = jnp.zeros((16,), jnp.float32)

        pltpu.sync_copy(stage_v, acc_sh.at[pl.ds(s * SLICE, SLICE)])
        plsc.subcore_barrier()
        si_dma.wait()
        di_dma.wait()

        @pl.loop(0, TPW, step=NB)
        def _(j0):
            gds = [pltpu.async_copy(tab_hbm.at[si_v.at[j0 + b]],
                                    rows_v.at[b], gsems[b])
                   for b in range(NB)]
            sds = []
            for b in range(NB):
                gds[b].wait()
                sds.append(pltpu.async_copy(rows_v.at[b],
                                            acc_sh.at[di_v.at[j0 + b]],
                                            ssems[b], add=True))
            for d in sds:
                d.wait()

        plsc.subcore_barrier()
        pltpu.sync_copy(acc_sh.at[pl.ds(s * SLICE, SLICE)], stage_v)
        pltpu.sync_copy(stage_v, out_hbm.at[c, pl.ds(s * SLICE, SLICE)])

    return _prop


_prop16 = _make_prop(H1)
_prop32 = _make_prop(H2)


R16 = N * H1 // 128
R32 = N * H2 // 128


def _expand_dinv(dinv_l, rep, rows):
    width = 128 * rep
    l_i = lax.broadcasted_iota(jnp.int32, (128, width), 0)
    c_i = lax.broadcasted_iota(jnp.int32, (128, width), 1)
    m = (l_i == c_i // rep).astype(jnp.float32)
    e = jnp.dot(dinv_l, m, preferred_element_type=jnp.float32)
    return jnp.reshape(e, (NPAD * rep // 128, 128))[:rows]


def _edges_body(ei_ref, s_ref, d_ref):
    s_ref[:ER] = jnp.reshape(ei_ref[0:1, :], (ER, 128))
    d_ref[:ER] = jnp.reshape(ei_ref[1:2, :], (ER, 128))
    pi = lax.broadcasted_iota(jnp.int32, (EPR - ER, 128), 0) * 128
    pad = N + (pi + lax.broadcasted_iota(jnp.int32, (EPR - ER, 128), 1)) % (
        NPAD - N)
    s_ref[ER:] = pad
    d_ref[ER:] = pad


def _pre_body(deg_ref, x_ref, w1_ref, hs1_ref, de16_ref, de32_ref):
    deg = deg_ref[0] + deg_ref[1] + 1.0
    dinv_l = lax.rsqrt(deg)
    e16 = _expand_dinv(dinv_l, H1, R16)
    e32 = _expand_dinv(dinv_l, H2, R32)
    de16_ref[...] = e16
    de32_ref[...] = e32
    x8 = jnp.reshape(x_ref[...], (R16, 8 * D))
    w1r = jnp.reshape(jnp.broadcast_to(w1_ref[...][:, None, :],
                                       (D, 8, H1)), (D, 128))
    w1big = jnp.concatenate([w1r] * 8, axis=0)
    r_i = lax.broadcasted_iota(jnp.int32, (8 * D, 128), 0)
    c_i = lax.broadcasted_iota(jnp.int32, (8 * D, 128), 1)
    w1big = jnp.where(r_i // D == c_i // H1, w1big, 0.0)
    h_p = jnp.dot(x8, w1big, preferred_element_type=jnp.float32)
    hs1_ref[:R16] = h_p * e16
    hs1_ref[R16:] = jnp.zeros((NPAD * H1 // 128 - R16, 128), jnp.float32)


def _layer_body(a_ref, hs1_ref, de16_ref, de32_ref, w2_ref, b1_ref, o_ref):
    accp = a_ref[0, :R16] + a_ref[1, :R16] + hs1_ref[:R16]
    b1t = jnp.reshape(jnp.broadcast_to(b1_ref[...][:, None, :],
                                       (1, 128 // H1, H1)), (1, 128))
    z1p = jnp.maximum(accp * de16_ref[...] + b1t, 0.0)
    w2r = jnp.reshape(jnp.broadcast_to(w2_ref[...][:, None, :],
                                       (H1, 256 // H2, H2)), (H1, 256))
    w2big = jnp.concatenate([w2r] * (128 // H1), axis=0)
    r_i = lax.broadcasted_iota(jnp.int32, (128, 256), 0)
    c_i = lax.broadcasted_iota(jnp.int32, (128, 256), 1)
    w2big = jnp.where(r_i // H1 == c_i // H2, w2big, 0.0)
    h2 = jnp.dot(z1p, w2big, preferred_element_type=jnp.float32)
    h2p = jnp.reshape(h2, (R32, 128))
    o_ref[:R32] = h2p * de32_ref[...]
    o_ref[R32:] = jnp.zeros((NPAD * H2 // 128 - R32, 128), jnp.float32)


def _final_body(a_ref, hs2_ref, de32_ref, b2_ref, batch_ref, wo_ref,
                bo_ref, o_ref):
    accp = a_ref[0, :R32] + a_ref[1, :R32] + hs2_ref[:R32]
    b2t = jnp.reshape(jnp.broadcast_to(b2_ref[...][:, None, :],
                                       (1, 128 // H2, H2)), (1, 128))
    z2p = jnp.maximum(accp * de32_ref[...] + b2t, 0.0)
    gid = lax.broadcasted_iota(jnp.int32, (NUM_GRAPHS, R32), 0)
    sums = jnp.zeros((NUM_GRAPHS, H2), jnp.float32)
    cnts = jnp.zeros((NUM_GRAPHS, 1), jnp.float32)
    for b in range(4):
        pb = (batch_ref[b:b + 1, :] == gid).astype(jnp.float32)
        sb = jnp.dot(pb, z2p, preferred_element_type=jnp.float32)
        sums = sums + sb[:, b * H2:(b + 1) * H2]
        cnts = cnts + jnp.sum(pb, axis=1, keepdims=True)
    g = sums / jnp.maximum(cnts, 1.0)
    o_ref[...] = jnp.dot(g, wo_ref[...],
                         preferred_element_type=jnp.float32) + bo_ref[...]


def _tc(body, out_shape, *args):
    return pl.pallas_call(body, out_shape=out_shape)(*args)


def kernel(x, edge_index, batch, W1, b1, W2, b2, W_out, b_out):
    f32 = jnp.float32
    sd = jax.ShapeDtypeStruct
    srcl, dstl = _tc(
        _edges_body,
        (sd((EPR, 128), jnp.int32), sd((EPR, 128), jnp.int32)),
        edge_index)
    src_p = srcl.reshape(NTILE, TPW, CH)
    dst_p = dstl.reshape(NTILE, TPW, CH)
    batch_s = jnp.transpose(batch.reshape(R32, 4))

    deg2 = _deg_sc(dst_p)
    hs1_lin, de16, de32 = _tc(
        _pre_body,
        (sd((NPAD * H1 // 128, 128), f32), sd((R16, 128), f32),
         sd((R32, 128), f32)),
        deg2.reshape(NCORE, NPAD // 128, 128), x, W1)
    acc1 = _prop16(hs1_lin.reshape(NPAD, H1), src_p, dst_p)
    hs2_lin = _tc(
        _layer_body, sd((NPAD * H2 // 128, 128), f32),
        acc1.reshape(NCORE, NPAD * H1 // 128, 128), hs1_lin, de16, de32,
        W2, b1.reshape(1, H1))
    acc2 = _prop32(hs2_lin.reshape(NPAD, H2), src_p, dst_p)
    out = _tc(
        _final_body, sd((NUM_GRAPHS, NUM_CLASSES), f32),
        acc2.reshape(NCORE, NPAD * H2 // 128, 128), hs2_lin, de32,
        b2.reshape(1, H2), batch_s, W_out,
        b_out.reshape(1, NUM_CLASSES))
    return out
